# baseline (speedup 1.0000x reference)
import functools

import jax
import jax.numpy as jnp
from jax import lax
from jax.experimental import pallas as pl
from jax.experimental.pallas import tpu as pltpu
from jax.experimental.pallas import tpu_sc as plsc

A4 = 0.1
TEMP = 1.5
EPS = 1e-8

_NC = 2
_NS = 16
_NW = _NC * _NS


def _norm_body(l3_ref, l6_ref, out_ref):
    y = l3_ref[...] + A4 * l6_ref[...]
    ss = jnp.sum(y * y, axis=1, keepdims=True)
    n = jnp.maximum(jnp.sqrt(ss), EPS)
    out_ref[...] = (y / n).astype(jnp.bfloat16)


def _normalize(l3p, l6p):
    npad, d = l3p.shape
    br = 256
    return pl.pallas_call(
        _norm_body,
        grid=(npad // br,),
        in_specs=[
            pl.BlockSpec((br, d), lambda i: (i, 0)),
            pl.BlockSpec((br, d), lambda i: (i, 0)),
        ],
        out_specs=pl.BlockSpec((br, d), lambda i: (i, 0)),
        out_shape=jax.ShapeDtypeStruct((npad, d), jnp.bfloat16),
    )(l3p, l6p)


def _sc_gather(table, idx):
    b = idx.shape[0]
    d = table.shape[1]
    bpw = b // _NW
    k = 128
    nchunks = bpw // k
    mesh = plsc.VectorSubcoreMesh(core_axis_name="c", subcore_axis_name="s")

    @functools.partial(
        pl.kernel,
        mesh=mesh,
        out_type=jax.ShapeDtypeStruct((b, d), table.dtype),
        scratch_types=[
            pltpu.VMEM((bpw,), jnp.int32),
            pltpu.VMEM((k, d), table.dtype),
            pltpu.VMEM((k, d), table.dtype),
            pltpu.SemaphoreType.DMA,
            pltpu.SemaphoreType.DMA,
        ],
    )
    def gather_kernel(table_hbm, idx_hbm, out_hbm, idx_v, rows0, rows1, s0, s1):
        wid = lax.axis_index("s") * _NC + lax.axis_index("c")
        base = wid * bpw
        pltpu.sync_copy(idx_hbm.at[pl.ds(base, bpw)], idx_v)
        bufs = (rows0, rows1)
        sems = (s0, s1)
        handles = [None] * nchunks
        handles[0] = pltpu.async_copy(
            table_hbm.at[idx_v.at[pl.ds(0, k)]], bufs[0], sems[0]
        )
        for c in range(nchunks):
            if c + 1 < nchunks:
                handles[c + 1] = pltpu.async_copy(
                    table_hbm.at[idx_v.at[pl.ds((c + 1) * k, k)]],
                    bufs[(c + 1) % 2],
                    sems[(c + 1) % 2],
                )
            handles[c].wait()
            pltpu.sync_copy(bufs[c % 2], out_hbm.at[pl.ds(base + c * k, k)])

    return gather_kernel(table, idx)


def _loss_body(fhat_ref, gath_ref, l1_ref, l4_ref, l2_ref, l5_ref, lbl_ref,
               out_ref, acc_ref, *, br, s, n_real):
    i = pl.program_id(0)

    @pl.when(i == 0)
    def _():
        acc_ref[0] = 0.0

    a = fhat_ref[...].astype(jnp.float32)
    g = gath_ref[...].astype(jnp.float32)
    sims = jnp.sum(a[:, None, :] * g, axis=2)
    exp_s = jnp.exp(sims) / TEMP
    j = lax.broadcasted_iota(jnp.int32, (br, s), 1)
    num = jnp.sum(jnp.where(j == 0, exp_s, 0.0), axis=1, keepdims=True)
    den = jnp.sum(jnp.where(j > 0, exp_s, 0.0), axis=1, keepdims=True)
    res = -jnp.log(num / den)
    row = i * br + lax.broadcasted_iota(jnp.int32, (br, 1), 0)
    acc_ref[0] += jnp.sum(jnp.where(row < n_real, res, 0.0))

    @pl.when(i == pl.num_programs(0) - 1)
    def _():
        x1 = l1_ref[...] + A4 * l4_ref[...]
        x2 = l2_ref[...] + A4 * l5_ref[...]
        z = lbl_ref[...]
        b1 = jnp.mean(jnp.maximum(x1, 0.0) - x1 * z
                      + jnp.log1p(jnp.exp(-jnp.abs(x1))))
        b2 = jnp.mean(jnp.maximum(x2, 0.0) - x2 * z
                      + jnp.log1p(jnp.exp(-jnp.abs(x2))))
        total = b1 + b2 + acc_ref[0] / n_real
        out_ref[...] = jnp.broadcast_to(total, (1, 1))


def _loss(fhat, gath3, l1, l4, l2, l5, lbl, n_real):
    npad, s, d = gath3.shape
    br = 256
    k2 = l1.shape[1]
    small = pl.BlockSpec((1, k2), lambda i: (0, 0))
    return pl.pallas_call(
        functools.partial(_loss_body, br=br, s=s, n_real=n_real),
        grid=(npad // br,),
        in_specs=[
            pl.BlockSpec((br, d), lambda i: (i, 0)),
            pl.BlockSpec((br, s, d), lambda i: (i, 0, 0)),
            small, small, small, small, small,
        ],
        out_specs=pl.BlockSpec((1, 1), lambda i: (0, 0)),
        out_shape=jax.ShapeDtypeStruct((1, 1), jnp.float32),
        scratch_shapes=[pltpu.SMEM((1,), jnp.float32)],
    )(fhat, gath3, l1, l4, l2, l5, lbl)


def kernel(logits1, logits2, logits3, logits4, logits5, logits6, lbl, sample):
    n, d = logits3.shape
    s = sample.shape[1]
    npad = ((n + 319) // 320) * 320
    l3p = jnp.pad(logits3, ((0, npad - n), (0, 0)))
    l6p = jnp.pad(logits6, ((0, npad - n), (0, 0)))
    fhat = _normalize(l3p, l6p)
    samp = jnp.pad(sample, ((0, npad - n), (0, 0))).astype(jnp.int32)
    idx = samp.reshape(-1)
    tab32 = lax.bitcast_convert_type(fhat.reshape(npad, d // 2, 2), jnp.int32)
    g32 = _sc_gather(tab32, idx)
    gath = lax.bitcast_convert_type(g32, jnp.bfloat16)
    gath3 = gath.reshape(npad, s, d)
    out = _loss(fhat, gath3, logits1, logits4, logits2, logits5, lbl, n)
    return out[0, 0]


# speedup vs baseline: 12.5850x; 12.5850x over previous
import functools

import jax
import jax.numpy as jnp
import numpy as np
from jax import lax
from jax.experimental import pallas as pl
from jax.experimental.pallas import tpu as pltpu
from jax.experimental.pallas import tpu_sc as plsc

A4 = 0.1
TEMP = 1.5
EPS = 1e-8

_NC = 2
_NS = 16
_NW = _NC * _NS

_HI = np.uint32(0xFFFF0000)


def _bf16_bits(x):
    bits = lax.bitcast_convert_type(x, jnp.uint32)
    lsb = (bits >> 16) & np.uint32(1)
    return (bits + np.uint32(0x7FFF) + lsb) & _HI


def _norm_pack_body(l3_ref, l6_ref, out_ref, *, dh):
    y = l3_ref[...] + A4 * l6_ref[...]
    ss = jnp.sum(y * y, axis=1, keepdims=True)
    n = jnp.maximum(jnp.sqrt(ss), EPS)
    f = y / n
    hi = _bf16_bits(f[:, :dh])
    lo = _bf16_bits(f[:, dh:])
    out_ref[...] = lax.bitcast_convert_type(hi | (lo >> 16), jnp.int32)


def _normalize_pack(l3p, l6p):
    npad, d = l3p.shape
    dh = d // 2
    br = 256
    return pl.pallas_call(
        functools.partial(_norm_pack_body, dh=dh),
        grid=(npad // br,),
        in_specs=[
            pl.BlockSpec((br, d), lambda i: (i, 0)),
            pl.BlockSpec((br, d), lambda i: (i, 0)),
        ],
        out_specs=pl.BlockSpec((br, dh), lambda i: (i, 0)),
        out_shape=jax.ShapeDtypeStruct((npad, dh), jnp.int32),
    )(l3p, l6p)


def _sc_gather(table, idx):
    b = idx.shape[0]
    dh = table.shape[1]
    bpw = b // _NW
    k = 128
    nchunks = bpw // k
    mesh = plsc.VectorSubcoreMesh(core_axis_name="c", subcore_axis_name="s")

    @functools.partial(
        pl.kernel,
        mesh=mesh,
        out_type=jax.ShapeDtypeStruct((b, dh), jnp.int32),
        scratch_types=[
            pltpu.VMEM((bpw,), jnp.int32),
            pltpu.VMEM((k, dh), jnp.int32),
            pltpu.VMEM((k, dh), jnp.int32),
            pltpu.SemaphoreType.DMA,
            pltpu.SemaphoreType.DMA,
        ],
    )
    def gather_kernel(table_hbm, idx_hbm, out_hbm, idx_v, rows0, rows1, s0, s1):
        wid = lax.axis_index("s") * _NC + lax.axis_index("c")
        base = wid * bpw
        pltpu.sync_copy(idx_hbm.at[pl.ds(base, bpw)], idx_v)
        bufs = (rows0, rows1)
        sems = (s0, s1)
        handles = [None] * nchunks
        handles[0] = pltpu.async_copy(
            table_hbm.at[idx_v.at[pl.ds(0, k)]], bufs[0], sems[0]
        )
        for c in range(nchunks):
            if c + 1 < nchunks:
                handles[c + 1] = pltpu.async_copy(
                    table_hbm.at[idx_v.at[pl.ds((c + 1) * k, k)]],
                    bufs[(c + 1) % 2],
                    sems[(c + 1) % 2],
                )
            handles[c].wait()
            pltpu.sync_copy(bufs[c % 2], out_hbm.at[pl.ds(base + c * k, k)])

    return gather_kernel(table, idx)


def _unpack(u):
    uu = lax.bitcast_convert_type(u, jnp.uint32)
    hi = lax.bitcast_convert_type(uu & _HI, jnp.float32)
    lo = lax.bitcast_convert_type(uu << 16, jnp.float32)
    return hi, lo


def _loss_body(fhat_ref, gath_ref, l1_ref, l4_ref, l2_ref, l5_ref, lbl_ref,
               out_ref, acc_ref, *, br, s, n_real):
    i = pl.program_id(0)

    @pl.when(i == 0)
    def _():
        acc_ref[0] = 0.0

    ahi, alo = _unpack(fhat_ref[...])
    ghi, glo = _unpack(gath_ref[...])
    sims = jnp.sum(ahi[:, None, :] * ghi + alo[:, None, :] * glo, axis=2)
    exp_s = jnp.exp(sims) / TEMP
    j = lax.broadcasted_iota(jnp.int32, (br, s), 1)
    num = jnp.sum(jnp.where(j == 0, exp_s, 0.0), axis=1, keepdims=True)
    den = jnp.sum(jnp.where(j > 0, exp_s, 0.0), axis=1, keepdims=True)
    res = -jnp.log(num / den)
    row = i * br + lax.broadcasted_iota(jnp.int32, (br, 1), 0)
    acc_ref[0] += jnp.sum(jnp.where(row < n_real, res, 0.0))

    @pl.when(i == pl.num_programs(0) - 1)
    def _():
        x1 = l1_ref[...] + A4 * l4_ref[...]
        x2 = l2_ref[...] + A4 * l5_ref[...]
        z = lbl_ref[...]
        b1 = jnp.mean(jnp.maximum(x1, 0.0) - x1 * z
                      + jnp.log1p(jnp.exp(-jnp.abs(x1))))
        b2 = jnp.mean(jnp.maximum(x2, 0.0) - x2 * z
                      + jnp.log1p(jnp.exp(-jnp.abs(x2))))
        total = b1 + b2 + acc_ref[0] / n_real
        out_ref[...] = jnp.broadcast_to(total, (1, 1))


def _loss(fpack, gath3, l1, l4, l2, l5, lbl, n_real):
    npad, s, dh = gath3.shape
    br = 256
    k2 = l1.shape[1]
    small = pl.BlockSpec((1, k2), lambda i: (0, 0))
    return pl.pallas_call(
        functools.partial(_loss_body, br=br, s=s, n_real=n_real),
        grid=(npad // br,),
        in_specs=[
            pl.BlockSpec((br, dh), lambda i: (i, 0)),
            pl.BlockSpec((br, s, dh), lambda i: (i, 0, 0)),
            small, small, small, small, small,
        ],
        out_specs=pl.BlockSpec((1, 1), lambda i: (0, 0)),
        out_shape=jax.ShapeDtypeStruct((1, 1), jnp.float32),
        scratch_shapes=[pltpu.SMEM((1,), jnp.float32)],
    )(fpack, gath3, l1, l4, l2, l5, lbl)


def kernel(logits1, logits2, logits3, logits4, logits5, logits6, lbl, sample):
    n, d = logits3.shape
    s = sample.shape[1]
    npad = ((n + 319) // 320) * 320
    l3p = jnp.pad(logits3, ((0, npad - n), (0, 0)))
    l6p = jnp.pad(logits6, ((0, npad - n), (0, 0)))
    fpack = _normalize_pack(l3p, l6p)
    samp = jnp.pad(sample, ((0, npad - n), (0, 0))).astype(jnp.int32)
    idx = samp.reshape(-1)
    g32 = _sc_gather(fpack, idx)
    gath3 = g32.reshape(npad, s, d // 2)
    out = _loss(fpack, gath3, logits1, logits4, logits2, logits5, lbl, n)
    return out[0, 0]


# speedup vs baseline: 18.9069x; 1.5023x over previous
import functools

import jax
import jax.numpy as jnp
import numpy as np
from jax import lax
from jax.experimental import pallas as pl
from jax.experimental.pallas import tpu as pltpu
from jax.experimental.pallas import tpu_sc as plsc

A4 = 0.1
TEMP = 1.5
EPS = 1e-8

_NC = 2
_NS = 16
_NW = _NC * _NS

_HI = np.uint32(0xFFFF0000)


def _bf16_bits(x):
    bits = lax.bitcast_convert_type(x, jnp.uint32)
    lsb = (bits >> 16) & np.uint32(1)
    return (bits + np.uint32(0x7FFF) + lsb) & _HI


def _norm_pack_body(l3_ref, l6_ref, out_ref, *, dh):
    y = l3_ref[...] + A4 * l6_ref[...]
    ss = jnp.sum(y * y, axis=1, keepdims=True)
    n = jnp.maximum(jnp.sqrt(ss), EPS)
    f = y / n
    hi = _bf16_bits(f[:, :dh])
    lo = _bf16_bits(f[:, dh:])
    out_ref[...] = lax.bitcast_convert_type(hi | (lo >> 16), jnp.int32)


def _normalize_pack(l3, l6, npad):
    n, d = l3.shape
    dh = d // 2
    br = 256
    return pl.pallas_call(
        functools.partial(_norm_pack_body, dh=dh),
        grid=(npad // br,),
        in_specs=[
            pl.BlockSpec((br, d), lambda i: (i, 0)),
            pl.BlockSpec((br, d), lambda i: (i, 0)),
        ],
        out_specs=pl.BlockSpec((br, dh), lambda i: (i, 0)),
        out_shape=jax.ShapeDtypeStruct((npad, dh), jnp.int32),
    )(l3, l6)


def _sc_gather(table, idx):
    b = idx.shape[0]
    dh = table.shape[1]
    bpw = b // _NW
    k = 128
    nchunks = bpw // k
    mesh = plsc.VectorSubcoreMesh(core_axis_name="c", subcore_axis_name="s")

    @functools.partial(
        pl.kernel,
        mesh=mesh,
        out_type=jax.ShapeDtypeStruct((b, dh), jnp.int32),
        scratch_types=[
            pltpu.VMEM((bpw,), jnp.int32),
            pltpu.VMEM((k, dh), jnp.int32),
            pltpu.VMEM((k, dh), jnp.int32),
            pltpu.SemaphoreType.DMA,
            pltpu.SemaphoreType.DMA,
        ],
    )
    def gather_kernel(table_hbm, idx_hbm, out_hbm, idx_v, rows0, rows1, s0, s1):
        wid = lax.axis_index("s") * _NC + lax.axis_index("c")
        base = wid * bpw
        pltpu.sync_copy(idx_hbm.at[pl.ds(base, bpw)], idx_v)
        bufs = (rows0, rows1)
        sems = (s0, s1)
        handles = [None] * nchunks
        handles[0] = pltpu.async_copy(
            table_hbm.at[idx_v.at[pl.ds(0, k)]], bufs[0], sems[0]
        )
        for c in range(nchunks):
            if c + 1 < nchunks:
                handles[c + 1] = pltpu.async_copy(
                    table_hbm.at[idx_v.at[pl.ds((c + 1) * k, k)]],
                    bufs[(c + 1) % 2],
                    sems[(c + 1) % 2],
                )
            handles[c].wait()
            pltpu.sync_copy(bufs[c % 2], out_hbm.at[pl.ds(base + c * k, k)])

    return gather_kernel(table, idx)


def _unpack(u):
    uu = lax.bitcast_convert_type(u, jnp.uint32)
    hi = lax.bitcast_convert_type(uu & _HI, jnp.float32)
    lo = lax.bitcast_convert_type(uu << 16, jnp.float32)
    return hi, lo


def _loss_body(fhat_ref, gath_ref, l1_ref, l4_ref, l2_ref, l5_ref, lbl_ref,
               out_ref, acc_ref, *, br, s, n_real):
    i = pl.program_id(0)

    @pl.when(i == 0)
    def _():
        acc_ref[0] = 0.0

    ahi, alo = _unpack(fhat_ref[...])
    ghi, glo = _unpack(gath_ref[...])
    sims = jnp.sum(ahi[None] * ghi + alo[None] * glo, axis=2)
    exp_s = jnp.exp(sims) / TEMP
    j = lax.broadcasted_iota(jnp.int32, (s, br), 0)
    num = jnp.sum(jnp.where(j == 0, exp_s, 0.0), axis=0, keepdims=True)
    den = jnp.sum(jnp.where(j > 0, exp_s, 0.0), axis=0, keepdims=True)
    res = -jnp.log(num / den)
    row = i * br + lax.broadcasted_iota(jnp.int32, (1, br), 1)
    acc_ref[0] += jnp.sum(jnp.where(row < n_real, res, 0.0))

    @pl.when(i == pl.num_programs(0) - 1)
    def _():
        x1 = l1_ref[...] + A4 * l4_ref[...]
        x2 = l2_ref[...] + A4 * l5_ref[...]
        z = lbl_ref[...]
        b1 = jnp.mean(jnp.maximum(x1, 0.0) - x1 * z
                      + jnp.log1p(jnp.exp(-jnp.abs(x1))))
        b2 = jnp.mean(jnp.maximum(x2, 0.0) - x2 * z
                      + jnp.log1p(jnp.exp(-jnp.abs(x2))))
        total = b1 + b2 + acc_ref[0] / n_real
        out_ref[...] = jnp.broadcast_to(total, (1, 1))


def _loss(fpack, gath3, l1, l4, l2, l5, lbl, n_real):
    s, npad, dh = gath3.shape
    br = 256
    k2 = l1.shape[1]
    small = pl.BlockSpec((1, k2), lambda i: (0, 0))
    return pl.pallas_call(
        functools.partial(_loss_body, br=br, s=s, n_real=n_real),
        grid=(npad // br,),
        in_specs=[
            pl.BlockSpec((br, dh), lambda i: (i, 0)),
            pl.BlockSpec((s, br, dh), lambda i: (0, i, 0)),
            small, small, small, small, small,
        ],
        out_specs=pl.BlockSpec((1, 1), lambda i: (0, 0)),
        out_shape=jax.ShapeDtypeStruct((1, 1), jnp.float32),
        scratch_shapes=[pltpu.SMEM((1,), jnp.float32)],
    )(fpack, gath3, l1, l4, l2, l5, lbl)


def kernel(logits1, logits2, logits3, logits4, logits5, logits6, lbl, sample):
    n, d = logits3.shape
    s = sample.shape[1]
    npad = ((n + 319) // 320) * 320
    fpack = _normalize_pack(logits3, logits6, npad)
    samp = jnp.pad(sample, ((0, npad - n), (0, 0))).astype(jnp.int32)
    idx = samp.T.reshape(-1)
    g32 = _sc_gather(fpack, idx)
    gath3 = g32.reshape(s, npad, d // 2)
    out = _loss(fpack, gath3, logits1, logits4, logits2, logits5, lbl, n)
    return out[0, 0]


# speedup vs baseline: 20.4158x; 1.0798x over previous
import functools

import jax
import jax.numpy as jnp
import numpy as np
from jax import lax
from jax.experimental import pallas as pl
from jax.experimental.pallas import tpu as pltpu
from jax.experimental.pallas import tpu_sc as plsc

A4 = 0.1
TEMP = 1.5
EPS = 1e-8

_NC = 2
_NS = 16
_NW = _NC * _NS

_HI = np.uint32(0xFFFF0000)


def _bf16_bits(x):
    bits = lax.bitcast_convert_type(x, jnp.uint32)
    lsb = (bits >> 16) & np.uint32(1)
    return (bits + np.uint32(0x7FFF) + lsb) & _HI


def _norm_pack_body(l3_ref, l6_ref, out_ref, *, dh):
    y = l3_ref[...] + A4 * l6_ref[...]
    ss = jnp.sum(y * y, axis=1, keepdims=True)
    n = jnp.maximum(jnp.sqrt(ss), EPS)
    f = y / n
    hi = _bf16_bits(f[:, :dh])
    lo = _bf16_bits(f[:, dh:])
    out_ref[...] = lax.bitcast_convert_type(hi | (lo >> 16), jnp.int32)


def _normalize_pack(l3, l6, npad):
    n, d = l3.shape
    dh = d // 2
    br = 512
    return pl.pallas_call(
        functools.partial(_norm_pack_body, dh=dh),
        grid=(npad // br,),
        in_specs=[
            pl.BlockSpec((br, d), lambda i: (i, 0)),
            pl.BlockSpec((br, d), lambda i: (i, 0)),
        ],
        out_specs=pl.BlockSpec((br, dh), lambda i: (i, 0)),
        out_shape=jax.ShapeDtypeStruct((npad, dh), jnp.int32),
    )(l3, l6)


def _sc_gather(table, idx):
    b = idx.shape[0]
    dh = table.shape[1]
    bpw = b // _NW
    k = 80
    nbuf = 4
    nchunks = bpw // k
    mesh = plsc.VectorSubcoreMesh(core_axis_name="c", subcore_axis_name="s")

    @functools.partial(
        pl.kernel,
        mesh=mesh,
        out_type=jax.ShapeDtypeStruct((b, dh), jnp.int32),
        scratch_types=[
            pltpu.VMEM((bpw,), jnp.int32),
            [pltpu.VMEM((k, dh), jnp.int32) for _ in range(nbuf)],
            [pltpu.SemaphoreType.DMA for _ in range(nbuf)],
            [pltpu.SemaphoreType.DMA for _ in range(nbuf)],
        ],
    )
    def gather_kernel(table_hbm, idx_hbm, out_hbm, idx_v, bufs, gsems, wsems):
        wid = lax.axis_index("s") * _NC + lax.axis_index("c")
        base = wid * bpw
        pltpu.sync_copy(idx_hbm.at[pl.ds(base, bpw)], idx_v)

        ghand = [None] * nchunks
        whand = [None] * nchunks

        def gstart(c):
            bb = c % nbuf
            ghand[c] = pltpu.async_copy(
                table_hbm.at[idx_v.at[pl.ds(c * k, k)]], bufs[bb], gsems[bb]
            )

        def wstart(c):
            bb = c % nbuf
            whand[c] = pltpu.async_copy(
                bufs[bb], out_hbm.at[pl.ds(base + c * k, k)], wsems[bb]
            )

        for c in range(min(nbuf - 1, nchunks)):
            gstart(c)
        for c in range(nchunks):
            pre = c + nbuf - 1
            if pre < nchunks:
                if c >= 1:
                    whand[pre - nbuf].wait()
                gstart(pre)
            ghand[c].wait()
            wstart(c)
        for c in range(max(0, nchunks - nbuf), nchunks):
            if whand[c] is not None:
                whand[c].wait()

    return gather_kernel(table, idx)


def _unpack(u):
    uu = lax.bitcast_convert_type(u, jnp.uint32)
    hi = lax.bitcast_convert_type(uu & _HI, jnp.float32)
    lo = lax.bitcast_convert_type(uu << 16, jnp.float32)
    return hi, lo


def _loss_body(fhat_ref, gath_ref, l1_ref, l4_ref, l2_ref, l5_ref, lbl_ref,
               out_ref, acc_ref, *, br, s, n_real):
    i = pl.program_id(0)

    @pl.when(i == 0)
    def _():
        acc_ref[0] = 0.0

    ahi, alo = _unpack(fhat_ref[...])
    ghi, glo = _unpack(gath_ref[...])
    sims = jnp.sum(ahi[None] * ghi + alo[None] * glo, axis=2)
    exp_s = jnp.exp(sims) / TEMP
    j = lax.broadcasted_iota(jnp.int32, (s, br), 0)
    num = jnp.sum(jnp.where(j == 0, exp_s, 0.0), axis=0, keepdims=True)
    den = jnp.sum(jnp.where(j > 0, exp_s, 0.0), axis=0, keepdims=True)
    res = -jnp.log(num / den)
    row = i * br + lax.broadcasted_iota(jnp.int32, (1, br), 1)
    acc_ref[0] += jnp.sum(jnp.where(row < n_real, res, 0.0))

    @pl.when(i == pl.num_programs(0) - 1)
    def _():
        x1 = l1_ref[...] + A4 * l4_ref[...]
        x2 = l2_ref[...] + A4 * l5_ref[...]
        z = lbl_ref[...]
        b1 = jnp.mean(jnp.maximum(x1, 0.0) - x1 * z
                      + jnp.log1p(jnp.exp(-jnp.abs(x1))))
        b2 = jnp.mean(jnp.maximum(x2, 0.0) - x2 * z
                      + jnp.log1p(jnp.exp(-jnp.abs(x2))))
        total = b1 + b2 + acc_ref[0] / n_real
        out_ref[...] = jnp.broadcast_to(total, (1, 1))


def _loss(fpack, gath3, l1, l4, l2, l5, lbl, n_real):
    s, npad, dh = gath3.shape
    br = 512
    k2 = l1.shape[1]
    small = pl.BlockSpec((1, k2), lambda i: (0, 0))
    return pl.pallas_call(
        functools.partial(_loss_body, br=br, s=s, n_real=n_real),
        grid=(npad // br,),
        in_specs=[
            pl.BlockSpec((br, dh), lambda i: (i, 0)),
            pl.BlockSpec((s, br, dh), lambda i: (0, i, 0)),
            small, small, small, small, small,
        ],
        out_specs=pl.BlockSpec((1, 1), lambda i: (0, 0)),
        out_shape=jax.ShapeDtypeStruct((1, 1), jnp.float32),
        scratch_shapes=[pltpu.SMEM((1,), jnp.float32)],
    )(fpack, gath3, l1, l4, l2, l5, lbl)


def kernel(logits1, logits2, logits3, logits4, logits5, logits6, lbl, sample):
    n, d = logits3.shape
    s = sample.shape[1]
    npad = ((n + 319) // 320) * 320
    fpack = _normalize_pack(logits3, logits6, npad)
    samp = jnp.pad(sample, ((0, npad - n), (0, 0))).astype(jnp.int32)
    idx = samp.T.reshape(-1)
    g32 = _sc_gather(fpack, idx)
    gath3 = g32.reshape(s, npad, d // 2)
    out = _loss(fpack, gath3, logits1, logits4, logits2, logits5, lbl, n)
    return out[0, 0]


# speedup vs baseline: 22.2159x; 1.0882x over previous
import functools

import jax
import jax.numpy as jnp
import numpy as np
from jax import lax
from jax.experimental import pallas as pl
from jax.experimental.pallas import tpu as pltpu
from jax.experimental.pallas import tpu_sc as plsc

A4 = 0.1
TEMP = 1.5
EPS = 1e-8

_NC = 2
_NS = 16
_NW = _NC * _NS

_HI = np.uint32(0xFFFF0000)
_L = 16


def _bf16_bits(x):
    bits = lax.bitcast_convert_type(x, jnp.uint32)
    lsb = (bits >> 16) & np.uint32(1)
    return (bits + np.uint32(0x7FFF) + lsb) & _HI


def _norm_pack_body(l3_ref, l6_ref, out_ref, *, dh):
    y = l3_ref[...] + A4 * l6_ref[...]
    ss = jnp.sum(y * y, axis=1, keepdims=True)
    n = jnp.maximum(jnp.sqrt(ss), EPS)
    f = y / n
    hi = _bf16_bits(f[:, :dh])
    lo = _bf16_bits(f[:, dh:])
    out_ref[...] = lax.bitcast_convert_type(hi | (lo >> 16), jnp.int32)


def _normalize_pack(l3, l6, npad):
    n, d = l3.shape
    dh = d // 2
    br = 512
    return pl.pallas_call(
        functools.partial(_norm_pack_body, dh=dh),
        grid=(npad // br,),
        in_specs=[
            pl.BlockSpec((br, d), lambda i: (i, 0)),
            pl.BlockSpec((br, d), lambda i: (i, 0)),
        ],
        out_specs=pl.BlockSpec((br, dh), lambda i: (i, 0)),
        out_shape=jax.ShapeDtypeStruct((npad, dh), jnp.int32),
    )(l3, l6)


def _sc_gather_dot(table, idx, s):
    b = idx.shape[0]
    dh = table.shape[1]
    nv = dh // _L
    npad = b // s
    bpw = b // _NW
    apw = npad // _NW
    ka = 8
    k = ka * s
    nchunks = bpw // k
    nbuf = 2
    mesh = plsc.VectorSubcoreMesh(core_axis_name="c", subcore_axis_name="s")

    @functools.partial(
        pl.kernel,
        mesh=mesh,
        out_type=jax.ShapeDtypeStruct((b, _L), jnp.float32),
        scratch_types=[
            pltpu.VMEM((bpw,), jnp.int32),
            [pltpu.VMEM((ka, dh), jnp.int32) for _ in range(nbuf)],
            [pltpu.VMEM((k, dh), jnp.int32) for _ in range(nbuf)],
            [pltpu.VMEM((k, _L), jnp.float32) for _ in range(nbuf)],
            [pltpu.SemaphoreType.DMA for _ in range(nbuf)],
            [pltpu.SemaphoreType.DMA for _ in range(nbuf)],
            [pltpu.SemaphoreType.DMA for _ in range(nbuf)],
        ],
    )
    def gd_kernel(table_hbm, idx_hbm, out_hbm, idx_v, abufs, bufs, pbufs,
                  asems, gsems, wsems):
        wid = lax.axis_index("s") * _NC + lax.axis_index("c")
        base = wid * bpw
        abase = wid * apw
        pltpu.sync_copy(idx_hbm.at[pl.ds(base, bpw)], idx_v)

        def gstart(c, bb):
            pltpu.async_copy(
                table_hbm.at[idx_v.at[pl.ds(c * k, k)]], bufs[bb], gsems[bb]
            )
            pltpu.async_copy(
                table_hbm.at[pl.ds(abase + c * ka, ka)], abufs[bb], asems[bb]
            )

        def gwait(bb):
            pltpu.make_async_copy(
                table_hbm.at[pl.ds(0, k)], bufs[bb], gsems[bb]
            ).wait()
            pltpu.make_async_copy(
                table_hbm.at[pl.ds(0, ka)], abufs[bb], asems[bb]
            ).wait()

        def wwait(bb):
            pltpu.make_async_copy(
                out_hbm.at[pl.ds(0, k)], pbufs[bb], wsems[bb]
            ).wait()

        def unpack(v):
            uv = lax.bitcast_convert_type(v, jnp.uint32)
            hi = lax.bitcast_convert_type(uv & _HI, jnp.float32)
            lo = lax.bitcast_convert_type(uv << 16, jnp.float32)
            return hi, lo

        gstart(0, 0)
        gstart(1, 1)

        def chunk_compute(c, bb):
            def anchor_body(a, carry):
                ahis, alos = [], []
                for v in range(nv):
                    hi, lo = unpack(abufs[bb][a, pl.ds(v * _L, _L)])
                    ahis.append(hi)
                    alos.append(lo)
                for j in range(s):
                    acc0 = jnp.zeros((_L,), jnp.float32)
                    acc1 = jnp.zeros((_L,), jnp.float32)
                    acc2 = jnp.zeros((_L,), jnp.float32)
                    acc3 = jnp.zeros((_L,), jnp.float32)
                    for v in range(nv):
                        rhi, rlo = unpack(bufs[bb][a * s + j, pl.ds(v * _L, _L)])
                        if v % 2 == 0:
                            acc0 = acc0 + ahis[v] * rhi
                            acc1 = acc1 + alos[v] * rlo
                        else:
                            acc2 = acc2 + ahis[v] * rhi
                            acc3 = acc3 + alos[v] * rlo
                    pbufs[bb][j * ka + a, :] = (acc0 + acc1) + (acc2 + acc3)
                return carry

            lax.fori_loop(0, ka, anchor_body, 0)

        nouter = nchunks // nbuf

        def outer(g, carry):
            for bb in range(nbuf):
                c = g * nbuf + bb
                gwait(bb)

                @pl.when(g > 0)
                def _():
                    wwait(bb)

                chunk_compute(c, bb)

                @pl.when(c + nbuf < nchunks)
                def _():
                    gstart(c + nbuf, bb)

                for j in range(s):
                    pltpu.async_copy(
                        pbufs[bb].at[pl.ds(j * ka, ka)],
                        out_hbm.at[pl.ds(j * npad + abase + c * ka, ka)],
                        wsems[bb],
                    )
            return carry

        lax.fori_loop(0, nouter, outer, 0)
        for bb in range(nbuf):
            wwait(bb)

    return gd_kernel(table, idx)


def _loss2_body(p_ref, l1_ref, l4_ref, l2_ref, l5_ref, lbl_ref, out_ref,
                num_ref, den_ref, *, rows, s, n_real):
    j = pl.program_id(0)
    li = lax.broadcasted_iota(jnp.int32, (256, _L), 0)
    gi = lax.broadcasted_iota(jnp.int32, (256, _L), 1)
    seg = jnp.where(li // _L == gi, 1.0, 0.0)
    s16 = lax.dot_general(p_ref[0], seg, (((1,), (0,)), ((), ())),
                          preferred_element_type=jnp.float32)
    e = jnp.exp(s16) / TEMP

    @pl.when(j == 0)
    def _():
        num_ref[...] = e
        den_ref[...] = jnp.zeros_like(e)

    @pl.when(j > 0)
    def _():
        den_ref[...] += e

    @pl.when(j == s - 1)
    def _():
        res = -jnp.log(num_ref[...] / den_ref[...])
        aidx = (lax.broadcasted_iota(jnp.int32, (rows, _L), 0) * _L
                + lax.broadcasted_iota(jnp.int32, (rows, _L), 1))
        lp = jnp.sum(jnp.where(aidx < n_real, res, 0.0)) / n_real
        x1 = l1_ref[...] + A4 * l4_ref[...]
        x2 = l2_ref[...] + A4 * l5_ref[...]
        z = lbl_ref[...]
        b1 = jnp.mean(jnp.maximum(x1, 0.0) - x1 * z
                      + jnp.log1p(jnp.exp(-jnp.abs(x1))))
        b2 = jnp.mean(jnp.maximum(x2, 0.0) - x2 * z
                      + jnp.log1p(jnp.exp(-jnp.abs(x2))))
        out_ref[...] = jnp.broadcast_to(b1 + b2 + lp, (1, 1))


def _loss2(part3, l1, l4, l2, l5, lbl, n_real):
    s, rows, _ = part3.shape
    k2 = l1.shape[1]
    small = pl.BlockSpec((1, k2), lambda j: (0, 0))
    return pl.pallas_call(
        functools.partial(_loss2_body, rows=rows, s=s, n_real=n_real),
        grid=(s,),
        in_specs=[
            pl.BlockSpec((1, rows, 256), lambda j: (j, 0, 0)),
            small, small, small, small, small,
        ],
        out_specs=pl.BlockSpec((1, 1), lambda j: (0, 0)),
        out_shape=jax.ShapeDtypeStruct((1, 1), jnp.float32),
        scratch_shapes=[
            pltpu.VMEM((rows, _L), jnp.float32),
            pltpu.VMEM((rows, _L), jnp.float32),
        ],
    )(part3, l1, l4, l2, l5, lbl)


def kernel(logits1, logits2, logits3, logits4, logits5, logits6, lbl, sample):
    n, d = logits3.shape
    s = sample.shape[1]
    npad = ((n + 319) // 320) * 320
    fpack = _normalize_pack(logits3, logits6, npad)
    samp = jnp.pad(sample, ((0, npad - n), (0, 0))).astype(jnp.int32)
    idx = samp.reshape(-1)
    part = _sc_gather_dot(fpack, idx, s)
    part3 = part.reshape(s, npad // _L, d // 2)
    out = _loss2(part3, logits1, logits4, logits2, logits5, lbl, n)
    return out[0, 0]


# speedup vs baseline: 27.5299x; 1.2392x over previous
import functools

import jax
import jax.numpy as jnp
import numpy as np
from jax import lax
from jax.experimental import pallas as pl
from jax.experimental.pallas import tpu as pltpu
from jax.experimental.pallas import tpu_sc as plsc

A4 = 0.1
TEMP = 1.5
EPS = 1e-8

_NC = 2
_NS = 16
_NW = _NC * _NS

_HI = np.uint32(0xFFFF0000)
_L = 16


def _bf16_bits(x):
    bits = lax.bitcast_convert_type(x, jnp.uint32)
    lsb = (bits >> 16) & np.uint32(1)
    return (bits + np.uint32(0x7FFF) + lsb) & _HI


def _norm_pack_body(l3_ref, l6_ref, out_ref, *, dh):
    y = l3_ref[...] + A4 * l6_ref[...]
    ss = jnp.sum(y * y, axis=1, keepdims=True)
    n = jnp.maximum(jnp.sqrt(ss), EPS)
    f = y / n
    hi = _bf16_bits(f[:, :dh])
    lo = _bf16_bits(f[:, dh:])
    out_ref[...] = lax.bitcast_convert_type(hi | (lo >> 16), jnp.int32)


def _normalize_pack(l3, l6, npad):
    n, d = l3.shape
    dh = d // 2
    br = 1024
    return pl.pallas_call(
        functools.partial(_norm_pack_body, dh=dh),
        grid=(npad // br,),
        in_specs=[
            pl.BlockSpec((br, d), lambda i: (i, 0)),
            pl.BlockSpec((br, d), lambda i: (i, 0)),
        ],
        out_specs=pl.BlockSpec((br, dh), lambda i: (i, 0)),
        out_shape=jax.ShapeDtypeStruct((npad, dh), jnp.int32),
    )(l3, l6)


def _sc_gather_dot(table, idx, s):
    b = idx.shape[0]
    dh = table.shape[1]
    nv = dh // _L
    npad = b // s
    bpw = b // _NW
    apw = npad // _NW
    ka = 8
    k = ka * s
    nchunks = bpw // k
    nbuf = 2
    mesh = plsc.VectorSubcoreMesh(core_axis_name="c", subcore_axis_name="s")

    @functools.partial(
        pl.kernel,
        mesh=mesh,
        out_type=jax.ShapeDtypeStruct((b * _L,), jnp.float32),
        scratch_types=[
            pltpu.VMEM((bpw,), jnp.int32),
            [pltpu.VMEM((ka, dh), jnp.int32) for _ in range(nbuf)],
            [pltpu.VMEM((k, dh), jnp.int32) for _ in range(nbuf)],
            [pltpu.VMEM((k * _L,), jnp.float32) for _ in range(nbuf)],
            [pltpu.SemaphoreType.DMA for _ in range(nbuf)],
            [pltpu.SemaphoreType.DMA for _ in range(nbuf)],
            [pltpu.SemaphoreType.DMA for _ in range(nbuf)],
        ],
    )
    def gd_kernel(table_hbm, idx_hbm, out_hbm, idx_v, abufs, bufs, pbufs,
                  asems, gsems, wsems):
        wid = lax.axis_index("s") * _NC + lax.axis_index("c")
        base = wid * bpw
        abase = wid * apw
        pltpu.sync_copy(idx_hbm.at[pl.ds(base, bpw)], idx_v)

        def gstart(c, bb):
            pltpu.async_copy(
                table_hbm.at[idx_v.at[pl.ds(c * k, k)]], bufs[bb], gsems[bb]
            )
            pltpu.async_copy(
                table_hbm.at[pl.ds(abase + c * ka, ka)], abufs[bb], asems[bb]
            )

        def gwait(bb):
            pltpu.make_async_copy(
                table_hbm.at[pl.ds(0, k)], bufs[bb], gsems[bb]
            ).wait()
            pltpu.make_async_copy(
                table_hbm.at[pl.ds(0, ka)], abufs[bb], asems[bb]
            ).wait()

        def wwait(bb):
            pltpu.make_async_copy(
                out_hbm.at[pl.ds(0, k * _L)], pbufs[bb], wsems[bb]
            ).wait()

        def unpack(v):
            uv = lax.bitcast_convert_type(v, jnp.uint32)
            hi = lax.bitcast_convert_type(uv & _HI, jnp.float32)
            lo = lax.bitcast_convert_type(uv << 16, jnp.float32)
            return hi, lo

        gstart(0, 0)
        gstart(1, 1)

        def chunk_compute(c, bb):
            def anchor_body(a, carry):
                ahis, alos = [], []
                for v in range(nv):
                    hi, lo = unpack(abufs[bb][a, pl.ds(v * _L, _L)])
                    ahis.append(hi)
                    alos.append(lo)
                for j in range(s):
                    acc0 = jnp.zeros((_L,), jnp.float32)
                    acc1 = jnp.zeros((_L,), jnp.float32)
                    acc2 = jnp.zeros((_L,), jnp.float32)
                    acc3 = jnp.zeros((_L,), jnp.float32)
                    for v in range(nv):
                        rhi, rlo = unpack(bufs[bb][a * s + j, pl.ds(v * _L, _L)])
                        if v % 2 == 0:
                            acc0 = acc0 + ahis[v] * rhi
                            acc1 = acc1 + alos[v] * rlo
                        else:
                            acc2 = acc2 + ahis[v] * rhi
                            acc3 = acc3 + alos[v] * rlo
                    pbufs[bb][pl.ds((j * ka + a) * _L, _L)] = (
                        (acc0 + acc1) + (acc2 + acc3))
                return carry

            lax.fori_loop(0, ka, anchor_body, 0)

        nouter = nchunks // nbuf

        def outer(g, carry):
            for bb in range(nbuf):
                c = g * nbuf + bb
                gwait(bb)

                @pl.when(g > 0)
                def _():
                    wwait(bb)

                chunk_compute(c, bb)

                @pl.when(c + nbuf < nchunks)
                def _():
                    gstart(c + nbuf, bb)

                for j in range(s):
                    pltpu.async_copy(
                        pbufs[bb].at[pl.ds(j * ka * _L, ka * _L)],
                        out_hbm.at[
                            pl.ds((j * npad + abase + c * ka) * _L, ka * _L)],
                        wsems[bb],
                    )
            return carry

        lax.fori_loop(0, nouter, outer, 0)
        for bb in range(nbuf):
            wwait(bb)

    return gd_kernel(table, idx)


def _loss2_body(p_ref, l1_ref, l4_ref, l2_ref, l5_ref, lbl_ref, out_ref,
                num_ref, den_ref, *, rows, s, n_real):
    j = pl.program_id(0)
    li = lax.broadcasted_iota(jnp.int32, (256, _L), 0)
    gi = lax.broadcasted_iota(jnp.int32, (256, _L), 1)
    seg = jnp.where(li // _L == gi, 1.0, 0.0)
    s16 = lax.dot_general(p_ref[0], seg, (((1,), (0,)), ((), ())),
                          preferred_element_type=jnp.float32)
    e = jnp.exp(s16) / TEMP

    @pl.when(j == 0)
    def _():
        num_ref[...] = e
        den_ref[...] = jnp.zeros_like(e)

    @pl.when(j > 0)
    def _():
        den_ref[...] += e

    @pl.when(j == s - 1)
    def _():
        res = -jnp.log(num_ref[...] / den_ref[...])
        aidx = (lax.broadcasted_iota(jnp.int32, (rows, _L), 0) * _L
                + lax.broadcasted_iota(jnp.int32, (rows, _L), 1))
        lp = jnp.sum(jnp.where(aidx < n_real, res, 0.0)) / n_real
        x1 = l1_ref[...] + A4 * l4_ref[...]
        x2 = l2_ref[...] + A4 * l5_ref[...]
        z = lbl_ref[...]
        b1 = jnp.mean(jnp.maximum(x1, 0.0) - x1 * z
                      + jnp.log1p(jnp.exp(-jnp.abs(x1))))
        b2 = jnp.mean(jnp.maximum(x2, 0.0) - x2 * z
                      + jnp.log1p(jnp.exp(-jnp.abs(x2))))
        out_ref[...] = jnp.broadcast_to(b1 + b2 + lp, (1, 1))


def _loss2(part3, l1, l4, l2, l5, lbl, n_real):
    s, rows, _ = part3.shape
    k2 = l1.shape[1]
    small = pl.BlockSpec((1, k2), lambda j: (0, 0))
    return pl.pallas_call(
        functools.partial(_loss2_body, rows=rows, s=s, n_real=n_real),
        grid=(s,),
        in_specs=[
            pl.BlockSpec((1, rows, 256), lambda j: (j, 0, 0)),
            small, small, small, small, small,
        ],
        out_specs=pl.BlockSpec((1, 1), lambda j: (0, 0)),
        out_shape=jax.ShapeDtypeStruct((1, 1), jnp.float32),
        scratch_shapes=[
            pltpu.VMEM((rows, _L), jnp.float32),
            pltpu.VMEM((rows, _L), jnp.float32),
        ],
    )(part3, l1, l4, l2, l5, lbl)


def kernel(logits1, logits2, logits3, logits4, logits5, logits6, lbl, sample):
    n, d = logits3.shape
    s = sample.shape[1]
    npad = ((n + 319) // 320) * 320
    fpack = _normalize_pack(logits3, logits6, npad)
    samp = jnp.pad(sample, ((0, npad - n), (0, 0))).astype(jnp.int32)
    idx = samp.reshape(-1)
    part = _sc_gather_dot(fpack, idx, s)
    part3 = part.reshape(s, npad // _L, d // 2)
    out = _loss2(part3, logits1, logits4, logits2, logits5, lbl, n)
    return out[0, 0]


# speedup vs baseline: 27.6970x; 1.0061x over previous
import functools

import jax
import jax.numpy as jnp
import numpy as np
from jax import lax
from jax.experimental import pallas as pl
from jax.experimental.pallas import tpu as pltpu
from jax.experimental.pallas import tpu_sc as plsc

A4 = 0.1
TEMP = 1.5
EPS = 1e-8

_NC = 2
_NS = 16
_NW = _NC * _NS

_HI = np.uint32(0xFFFF0000)
_L = 16


def _bf16_bits(x):
    bits = lax.bitcast_convert_type(x, jnp.uint32)
    lsb = (bits >> 16) & np.uint32(1)
    return (bits + np.uint32(0x7FFF) + lsb) & _HI


def _norm_pack_body(l3_ref, l6_ref, out_ref, *, dh):
    y = l3_ref[...] + A4 * l6_ref[...]
    ss = jnp.sum(y * y, axis=1, keepdims=True)
    n = jnp.maximum(jnp.sqrt(ss), EPS)
    f = y / n
    hi = _bf16_bits(f[:, :dh])
    lo = _bf16_bits(f[:, dh:])
    out_ref[...] = lax.bitcast_convert_type(hi | (lo >> 16), jnp.int32)


def _normalize_pack(l3, l6, npad):
    n, d = l3.shape
    dh = d // 2
    br = 1024
    return pl.pallas_call(
        functools.partial(_norm_pack_body, dh=dh),
        grid=(npad // br,),
        in_specs=[
            pl.BlockSpec((br, d), lambda i: (i, 0)),
            pl.BlockSpec((br, d), lambda i: (i, 0)),
        ],
        out_specs=pl.BlockSpec((br, dh), lambda i: (i, 0)),
        out_shape=jax.ShapeDtypeStruct((npad, dh), jnp.int32),
    )(l3, l6)


def _sc_gather_dot(table, idx, s):
    b = idx.shape[0]
    dh = table.shape[1]
    nv = dh // _L
    npad = b // s
    bpw = b // _NW
    apw = npad // _NW
    ka = 8
    k = ka * s
    nchunks = bpw // k
    nbuf = 4
    mesh = plsc.VectorSubcoreMesh(core_axis_name="c", subcore_axis_name="s")

    @functools.partial(
        pl.kernel,
        mesh=mesh,
        out_type=jax.ShapeDtypeStruct((b * _L,), jnp.float32),
        scratch_types=[
            pltpu.VMEM((bpw,), jnp.int32),
            [pltpu.VMEM((ka, dh), jnp.int32) for _ in range(nbuf)],
            [pltpu.VMEM((k, dh), jnp.int32) for _ in range(nbuf)],
            [pltpu.VMEM((k * _L,), jnp.float32) for _ in range(nbuf)],
            [pltpu.SemaphoreType.DMA for _ in range(nbuf)],
            [pltpu.SemaphoreType.DMA for _ in range(nbuf)],
            [pltpu.SemaphoreType.DMA for _ in range(nbuf)],
        ],
    )
    def gd_kernel(table_hbm, idx_hbm, out_hbm, idx_v, abufs, bufs, pbufs,
                  asems, gsems, wsems):
        wid = lax.axis_index("s") * _NC + lax.axis_index("c")
        base = wid * bpw
        abase = wid * apw
        pltpu.sync_copy(idx_hbm.at[pl.ds(base, bpw)], idx_v)

        def gstart(c, bb):
            pltpu.async_copy(
                table_hbm.at[idx_v.at[pl.ds(c * k, k)]], bufs[bb], gsems[bb]
            )
            pltpu.async_copy(
                table_hbm.at[pl.ds(abase + c * ka, ka)], abufs[bb], asems[bb]
            )

        def gwait(bb):
            pltpu.make_async_copy(
                table_hbm.at[pl.ds(0, k)], bufs[bb], gsems[bb]
            ).wait()
            pltpu.make_async_copy(
                table_hbm.at[pl.ds(0, ka)], abufs[bb], asems[bb]
            ).wait()

        def wwait(bb):
            pltpu.make_async_copy(
                out_hbm.at[pl.ds(0, k * _L)], pbufs[bb], wsems[bb]
            ).wait()

        def unpack(v):
            uv = lax.bitcast_convert_type(v, jnp.uint32)
            hi = lax.bitcast_convert_type(uv & _HI, jnp.float32)
            lo = lax.bitcast_convert_type(uv << 16, jnp.float32)
            return hi, lo

        for _c in range(nbuf):
            gstart(_c, _c)

        def chunk_compute(c, bb):
            def anchor_body(a, carry):
                ahis, alos = [], []
                for v in range(nv):
                    hi, lo = unpack(abufs[bb][a, pl.ds(v * _L, _L)])
                    ahis.append(hi)
                    alos.append(lo)
                for j in range(s):
                    acc0 = jnp.zeros((_L,), jnp.float32)
                    acc1 = jnp.zeros((_L,), jnp.float32)
                    acc2 = jnp.zeros((_L,), jnp.float32)
                    acc3 = jnp.zeros((_L,), jnp.float32)
                    for v in range(nv):
                        rhi, rlo = unpack(bufs[bb][a * s + j, pl.ds(v * _L, _L)])
                        if v % 2 == 0:
                            acc0 = acc0 + ahis[v] * rhi
                            acc1 = acc1 + alos[v] * rlo
                        else:
                            acc2 = acc2 + ahis[v] * rhi
                            acc3 = acc3 + alos[v] * rlo
                    pbufs[bb][pl.ds((j * ka + a) * _L, _L)] = (
                        (acc0 + acc1) + (acc2 + acc3))
                return carry

            lax.fori_loop(0, ka, anchor_body, 0)

        nouter = nchunks // nbuf

        def outer(g, carry):
            for bb in range(nbuf):
                c = g * nbuf + bb
                gwait(bb)

                @pl.when(g > 0)
                def _():
                    wwait(bb)

                chunk_compute(c, bb)

                @pl.when(c + nbuf < nchunks)
                def _():
                    gstart(c + nbuf, bb)

                for j in range(s):
                    pltpu.async_copy(
                        pbufs[bb].at[pl.ds(j * ka * _L, ka * _L)],
                        out_hbm.at[
                            pl.ds((j * npad + abase + c * ka) * _L, ka * _L)],
                        wsems[bb],
                    )
            return carry

        lax.fori_loop(0, nouter, outer, 0)
        for bb in range(nbuf):
            wwait(bb)

    return gd_kernel(table, idx)


def _loss2_body(p_ref, l1_ref, l4_ref, l2_ref, l5_ref, lbl_ref, out_ref,
                num_ref, den_ref, *, rows, s, n_real):
    j = pl.program_id(0)
    li = lax.broadcasted_iota(jnp.int32, (256, _L), 0)
    gi = lax.broadcasted_iota(jnp.int32, (256, _L), 1)
    seg = jnp.where(li // _L == gi, 1.0, 0.0)
    s16 = lax.dot_general(p_ref[0], seg, (((1,), (0,)), ((), ())),
                          preferred_element_type=jnp.float32)
    e = jnp.exp(s16) / TEMP

    @pl.when(j == 0)
    def _():
        num_ref[...] = e
        den_ref[...] = jnp.zeros_like(e)

    @pl.when(j > 0)
    def _():
        den_ref[...] += e

    @pl.when(j == s - 1)
    def _():
        res = -jnp.log(num_ref[...] / den_ref[...])
        aidx = (lax.broadcasted_iota(jnp.int32, (rows, _L), 0) * _L
                + lax.broadcasted_iota(jnp.int32, (rows, _L), 1))
        lp = jnp.sum(jnp.where(aidx < n_real, res, 0.0)) / n_real
        x1 = l1_ref[...] + A4 * l4_ref[...]
        x2 = l2_ref[...] + A4 * l5_ref[...]
        z = lbl_ref[...]
        b1 = jnp.mean(jnp.maximum(x1, 0.0) - x1 * z
                      + jnp.log1p(jnp.exp(-jnp.abs(x1))))
        b2 = jnp.mean(jnp.maximum(x2, 0.0) - x2 * z
                      + jnp.log1p(jnp.exp(-jnp.abs(x2))))
        out_ref[...] = jnp.broadcast_to(b1 + b2 + lp, (1, 1))


def _loss2(part3, l1, l4, l2, l5, lbl, n_real):
    s, rows, _ = part3.shape
    k2 = l1.shape[1]
    small = pl.BlockSpec((1, k2), lambda j: (0, 0))
    return pl.pallas_call(
        functools.partial(_loss2_body, rows=rows, s=s, n_real=n_real),
        grid=(s,),
        in_specs=[
            pl.BlockSpec((1, rows, 256), lambda j: (j, 0, 0)),
            small, small, small, small, small,
        ],
        out_specs=pl.BlockSpec((1, 1), lambda j: (0, 0)),
        out_shape=jax.ShapeDtypeStruct((1, 1), jnp.float32),
        scratch_shapes=[
            pltpu.VMEM((rows, _L), jnp.float32),
            pltpu.VMEM((rows, _L), jnp.float32),
        ],
    )(part3, l1, l4, l2, l5, lbl)


def kernel(logits1, logits2, logits3, logits4, logits5, logits6, lbl, sample):
    n, d = logits3.shape
    s = sample.shape[1]
    npad = ((n + 319) // 320) * 320
    fpack = _normalize_pack(logits3, logits6, npad)
    samp = jnp.pad(sample, ((0, npad - n), (0, 0))).astype(jnp.int32)
    idx = samp.reshape(-1)
    part = _sc_gather_dot(fpack, idx, s)
    part3 = part.reshape(s, npad // _L, d // 2)
    out = _loss2(part3, logits1, logits4, logits2, logits5, lbl, n)
    return out[0, 0]
